# baseline (speedup 1.0000x reference)
import functools

import jax
import jax.numpy as jnp
from jax import lax
from jax.experimental import pallas as pl
from jax.experimental.pallas import tpu as pltpu
from jax.experimental.pallas import tpu_sc as plsc

N_GENES = 100000
DIM_GENE = 64
DIM_MODEL = 128
EPS = 1e-5

_ROW_BLK = 10000


def _transform_body(t_ref, w_ref, b_ref, g_ref, bb_ref, o_ref):
    e = t_ref[...]
    h = jnp.dot(e, w_ref[...], preferred_element_type=jnp.float32) + b_ref[...]
    mean = jnp.mean(h, axis=-1, keepdims=True)
    c = h - mean
    var = jnp.mean(c * c, axis=-1, keepdims=True)
    o_ref[...] = c * lax.rsqrt(var + EPS) * g_ref[...] + bb_ref[...]


def _transform_table(table, W2, b2, gamma, beta):
    return pl.pallas_call(
        _transform_body,
        grid=(N_GENES // _ROW_BLK,),
        in_specs=[
            pl.BlockSpec((_ROW_BLK, DIM_GENE), lambda i: (i, 0)),
            pl.BlockSpec((DIM_GENE, DIM_MODEL), lambda i: (0, 0)),
            pl.BlockSpec((1, DIM_MODEL), lambda i: (0, 0)),
            pl.BlockSpec((1, DIM_MODEL), lambda i: (0, 0)),
            pl.BlockSpec((1, DIM_MODEL), lambda i: (0, 0)),
        ],
        out_specs=pl.BlockSpec((_ROW_BLK, DIM_MODEL), lambda i: (i, 0)),
        out_shape=jax.ShapeDtypeStruct((N_GENES, DIM_MODEL), jnp.float32),
    )(table, W2, b2.reshape(1, DIM_MODEL), gamma.reshape(1, DIM_MODEL),
      beta.reshape(1, DIM_MODEL))


def _sc_gather(tbl, idx2d, n_rows):
    info = plsc.get_sparse_core_info()
    nw = info.num_cores * info.num_subcores
    rows_per_w = n_rows // nw
    n_ir = rows_per_w // 128
    ch = 256
    n_ch = rows_per_w // ch
    nbuf = 3
    lag = 1
    n_grp = (n_ch + nbuf - 1) // nbuf + 1
    mesh = plsc.VectorSubcoreMesh(core_axis_name="c", subcore_axis_name="s")

    @functools.partial(
        pl.kernel, mesh=mesh,
        out_type=jax.ShapeDtypeStruct((n_rows, DIM_MODEL), jnp.float32),
        scratch_types=[
            pltpu.VMEM((n_ir, 128), jnp.int32),
            pltpu.VMEM((nbuf, ch, DIM_MODEL), jnp.float32),
        ] + [pltpu.SemaphoreType.DMA] * (2 * nbuf),
    )
    def k(tbl_hbm, idx_hbm, out_hbm, idx_v, rows_v, *sems):
        sem_g = sems[:nbuf]
        sem_s = sems[nbuf:]
        wid = lax.axis_index("s") * info.num_cores + lax.axis_index("c")
        out0 = wid * rows_per_w

        pltpu.sync_copy(idx_hbm.at[pl.ds(wid * n_ir, n_ir), :], idx_v)

        def gather_descs(c, b):
            return [
                pltpu.make_async_copy(
                    tbl_hbm.at[idx_v.at[2 * c + j]],
                    rows_v.at[b].at[pl.ds(j * 128, 128), :],
                    sem_g[b])
                for j in range(2)
            ]

        def scatter_desc(c, b):
            return pltpu.make_async_copy(
                rows_v.at[b],
                out_hbm.at[pl.ds(out0 + c * ch, ch), :],
                sem_s[b])

        def group(i, carry):
            for b in range(nbuf):
                c = i * nbuf + b
                bs = (b + nbuf - lag) % nbuf

                @pl.when(jnp.logical_and(c >= nbuf, c - nbuf < n_ch))
                def _():
                    scatter_desc(c - nbuf, b).wait()

                @pl.when(c < n_ch)
                def _():
                    for d in gather_descs(c, b):
                        d.start()

                @pl.when(jnp.logical_and(c >= lag, c - lag < n_ch))
                def _():
                    for d in gather_descs(c - lag, bs):
                        d.wait()
                    scatter_desc(c - lag, bs).start()
            return carry

        lax.fori_loop(0, n_grp, group, 0)

    return k(tbl, idx2d)


def kernel(x, table, W2, b2, gamma, beta):
    B, L = x.shape
    t = _transform_table(table, W2, b2, gamma, beta)
    idx2d = x.reshape(-1, 128).astype(jnp.int32)
    out = _sc_gather(t, idx2d, B * L)
    return out.reshape(B, L, DIM_MODEL)


# speedup vs baseline: 1.0744x; 1.0744x over previous
import functools

import jax
import jax.numpy as jnp
from jax import lax
from jax.experimental import pallas as pl
from jax.experimental.pallas import tpu as pltpu
from jax.experimental.pallas import tpu_sc as plsc

N_GENES = 100000
DIM_GENE = 64
DIM_MODEL = 128
EPS = 1e-5

_ROW_BLK = 4096
_N_PAD = 102400


def _transform_body(tT_ref, w_ref, b_ref, g_ref, bb_ref, o_ref):
    e_t = tT_ref[...]
    h = lax.dot_general(e_t, w_ref[...], (((0,), (0,)), ((), ())),
                        preferred_element_type=jnp.float32) + b_ref[...]
    mean = jnp.mean(h, axis=-1, keepdims=True)
    c = h - mean
    var = jnp.mean(c * c, axis=-1, keepdims=True)
    o_ref[...] = c * lax.rsqrt(var + EPS) * g_ref[...] + bb_ref[...]


def _transform_table(table, W2, b2, gamma, beta):
    tT = table.T
    return pl.pallas_call(
        _transform_body,
        grid=(_N_PAD // _ROW_BLK,),
        in_specs=[
            pl.BlockSpec((DIM_GENE, _ROW_BLK), lambda i: (0, i)),
            pl.BlockSpec((DIM_GENE, DIM_MODEL), lambda i: (0, 0)),
            pl.BlockSpec((1, DIM_MODEL), lambda i: (0, 0)),
            pl.BlockSpec((1, DIM_MODEL), lambda i: (0, 0)),
            pl.BlockSpec((1, DIM_MODEL), lambda i: (0, 0)),
        ],
        out_specs=pl.BlockSpec((_ROW_BLK, DIM_MODEL), lambda i: (i, 0)),
        out_shape=jax.ShapeDtypeStruct((_N_PAD, DIM_MODEL), jnp.float32),
    )(tT, W2, b2.reshape(1, DIM_MODEL), gamma.reshape(1, DIM_MODEL),
      beta.reshape(1, DIM_MODEL))


def _sc_gather(tbl, idx2d, n_rows):
    info = plsc.get_sparse_core_info()
    nw = info.num_cores * info.num_subcores
    rows_per_w = n_rows // nw
    n_ir = rows_per_w // 128
    ch = 256
    n_ch = rows_per_w // ch
    nbuf = 3
    lag = 1
    n_grp = (n_ch + nbuf - 1) // nbuf + 1
    mesh = plsc.VectorSubcoreMesh(core_axis_name="c", subcore_axis_name="s")

    @functools.partial(
        pl.kernel, mesh=mesh,
        out_type=jax.ShapeDtypeStruct((n_rows, DIM_MODEL), jnp.float32),
        scratch_types=[
            pltpu.VMEM((n_ir, 128), jnp.int32),
            pltpu.VMEM((nbuf, ch, DIM_MODEL), jnp.float32),
        ] + [pltpu.SemaphoreType.DMA] * (2 * nbuf),
    )
    def k(tbl_hbm, idx_hbm, out_hbm, idx_v, rows_v, *sems):
        sem_g = sems[:nbuf]
        sem_s = sems[nbuf:]
        wid = lax.axis_index("s") * info.num_cores + lax.axis_index("c")
        out0 = wid * rows_per_w

        pltpu.sync_copy(idx_hbm.at[pl.ds(wid * n_ir, n_ir), :], idx_v)

        def gather_descs(c, b):
            return [
                pltpu.make_async_copy(
                    tbl_hbm.at[idx_v.at[2 * c + j]],
                    rows_v.at[b].at[pl.ds(j * 128, 128), :],
                    sem_g[b])
                for j in range(2)
            ]

        def scatter_desc(c, b):
            return pltpu.make_async_copy(
                rows_v.at[b],
                out_hbm.at[pl.ds(out0 + c * ch, ch), :],
                sem_s[b])

        def group(i, carry):
            for b in range(nbuf):
                c = i * nbuf + b
                bs = (b + nbuf - lag) % nbuf

                @pl.when(jnp.logical_and(c >= nbuf, c - nbuf < n_ch))
                def _():
                    scatter_desc(c - nbuf, b).wait()

                @pl.when(c < n_ch)
                def _():
                    for d in gather_descs(c, b):
                        d.start()

                @pl.when(jnp.logical_and(c >= lag, c - lag < n_ch))
                def _():
                    for d in gather_descs(c - lag, bs):
                        d.wait()
                    scatter_desc(c - lag, bs).start()
            return carry

        lax.fori_loop(0, n_grp, group, 0)

    return k(tbl, idx2d)


def kernel(x, table, W2, b2, gamma, beta):
    B, L = x.shape
    t = _transform_table(table, W2, b2, gamma, beta)
    idx2d = x.reshape(-1, 128).astype(jnp.int32)
    out = _sc_gather(t, idx2d, B * L)
    return out.reshape(B, L, DIM_MODEL)


# speedup vs baseline: 1.0812x; 1.0063x over previous
import functools

import jax
import jax.numpy as jnp
from jax import lax
from jax.experimental import pallas as pl
from jax.experimental.pallas import tpu as pltpu
from jax.experimental.pallas import tpu_sc as plsc

N_GENES = 100000
DIM_GENE = 64
DIM_MODEL = 128
EPS = 1e-5

_ROW_BLK = 12800
_N_PAD = 102400


def _transform_body(tT_ref, w_ref, b_ref, g_ref, bb_ref, o_ref):
    e_t = tT_ref[...]
    h = lax.dot_general(e_t, w_ref[...], (((0,), (0,)), ((), ())),
                        preferred_element_type=jnp.float32) + b_ref[...]
    mean = jnp.mean(h, axis=-1, keepdims=True)
    c = h - mean
    var = jnp.mean(c * c, axis=-1, keepdims=True)
    o_ref[...] = c * lax.rsqrt(var + EPS) * g_ref[...] + bb_ref[...]


def _transform_table(table, W2, b2, gamma, beta):
    tT = table.T
    return pl.pallas_call(
        _transform_body,
        grid=(_N_PAD // _ROW_BLK,),
        in_specs=[
            pl.BlockSpec((DIM_GENE, _ROW_BLK), lambda i: (0, i)),
            pl.BlockSpec((DIM_GENE, DIM_MODEL), lambda i: (0, 0)),
            pl.BlockSpec((1, DIM_MODEL), lambda i: (0, 0)),
            pl.BlockSpec((1, DIM_MODEL), lambda i: (0, 0)),
            pl.BlockSpec((1, DIM_MODEL), lambda i: (0, 0)),
        ],
        out_specs=pl.BlockSpec((_ROW_BLK, DIM_MODEL), lambda i: (i, 0)),
        out_shape=jax.ShapeDtypeStruct((_N_PAD, DIM_MODEL), jnp.float32),
    )(tT, W2, b2.reshape(1, DIM_MODEL), gamma.reshape(1, DIM_MODEL),
      beta.reshape(1, DIM_MODEL))


def _sc_gather(tbl, idx2d, n_rows):
    info = plsc.get_sparse_core_info()
    nw = info.num_cores * info.num_subcores
    rows_per_w = n_rows // nw
    n_ir = rows_per_w // 128
    ch = 256
    n_ch = rows_per_w // ch
    nbuf = 3
    lag = 1
    n_grp = (n_ch + nbuf - 1) // nbuf + 1
    mesh = plsc.VectorSubcoreMesh(core_axis_name="c", subcore_axis_name="s")

    @functools.partial(
        pl.kernel, mesh=mesh,
        out_type=jax.ShapeDtypeStruct((n_rows, DIM_MODEL), jnp.float32),
        scratch_types=[
            pltpu.VMEM((n_ir, 128), jnp.int32),
            pltpu.VMEM((nbuf, ch, DIM_MODEL), jnp.float32),
        ] + [pltpu.SemaphoreType.DMA] * (2 * nbuf),
    )
    def k(tbl_hbm, idx_hbm, out_hbm, idx_v, rows_v, *sems):
        sem_g = sems[:nbuf]
        sem_s = sems[nbuf:]
        wid = lax.axis_index("s") * info.num_cores + lax.axis_index("c")
        out0 = wid * rows_per_w

        pltpu.sync_copy(idx_hbm.at[pl.ds(wid * n_ir, n_ir), :], idx_v)

        def gather_descs(c, b):
            return [
                pltpu.make_async_copy(
                    tbl_hbm.at[idx_v.at[2 * c + j]],
                    rows_v.at[b].at[pl.ds(j * 128, 128), :],
                    sem_g[b])
                for j in range(2)
            ]

        def scatter_desc(c, b):
            return pltpu.make_async_copy(
                rows_v.at[b],
                out_hbm.at[pl.ds(out0 + c * ch, ch), :],
                sem_s[b])

        def group(i, carry):
            for b in range(nbuf):
                c = i * nbuf + b
                bs = (b + nbuf - lag) % nbuf

                @pl.when(jnp.logical_and(c >= nbuf, c - nbuf < n_ch))
                def _():
                    scatter_desc(c - nbuf, b).wait()

                @pl.when(c < n_ch)
                def _():
                    for d in gather_descs(c, b):
                        d.start()

                @pl.when(jnp.logical_and(c >= lag, c - lag < n_ch))
                def _():
                    for d in gather_descs(c - lag, bs):
                        d.wait()
                    scatter_desc(c - lag, bs).start()
            return carry

        lax.fori_loop(0, n_grp, group, 0)

    return k(tbl, idx2d)


def kernel(x, table, W2, b2, gamma, beta):
    B, L = x.shape
    t = _transform_table(table, W2, b2, gamma, beta)
    idx2d = x.reshape(-1, 128).astype(jnp.int32)
    out = _sc_gather(t, idx2d, B * L)
    return out.reshape(B, L, DIM_MODEL)


# speedup vs baseline: 1.1090x; 1.0257x over previous
import functools

import jax
import jax.numpy as jnp
from jax import lax
from jax.experimental import pallas as pl
from jax.experimental.pallas import tpu as pltpu
from jax.experimental.pallas import tpu_sc as plsc

N_GENES = 100000
DIM_GENE = 64
DIM_MODEL = 128
EPS = 1e-5

_ROW_BLK = 12800
_N_PAD = 102400


def _transform_body(tT_ref, w_ref, b_ref, g_ref, bb_ref, o_ref):
    e_t = tT_ref[...]
    h = lax.dot_general(e_t, w_ref[...], (((0,), (0,)), ((), ())),
                        preferred_element_type=jnp.float32) + b_ref[...]
    ones_over = jnp.full((DIM_MODEL, DIM_MODEL), 1.0 / DIM_MODEL, jnp.float32)
    m = lax.dot_general(h, ones_over, (((1,), (0,)), ((), ())),
                        preferred_element_type=jnp.float32)
    m2 = lax.dot_general(h * h, ones_over, (((1,), (0,)), ((), ())),
                         preferred_element_type=jnp.float32)
    var = m2 - m * m
    o_ref[...] = (h - m) * lax.rsqrt(var + EPS) * g_ref[...] + bb_ref[...]


def _transform_table(table, W2, b2, gamma, beta):
    tT = table.T
    return pl.pallas_call(
        _transform_body,
        grid=(_N_PAD // _ROW_BLK,),
        in_specs=[
            pl.BlockSpec((DIM_GENE, _ROW_BLK), lambda i: (0, i)),
            pl.BlockSpec((DIM_GENE, DIM_MODEL), lambda i: (0, 0)),
            pl.BlockSpec((1, DIM_MODEL), lambda i: (0, 0)),
            pl.BlockSpec((1, DIM_MODEL), lambda i: (0, 0)),
            pl.BlockSpec((1, DIM_MODEL), lambda i: (0, 0)),
        ],
        out_specs=pl.BlockSpec((_ROW_BLK, DIM_MODEL), lambda i: (i, 0)),
        out_shape=jax.ShapeDtypeStruct((_N_PAD, DIM_MODEL), jnp.float32),
    )(tT, W2, b2.reshape(1, DIM_MODEL), gamma.reshape(1, DIM_MODEL),
      beta.reshape(1, DIM_MODEL))


def _sc_gather(tbl, idx2d, n_rows):
    info = plsc.get_sparse_core_info()
    nw = info.num_cores * info.num_subcores
    rows_per_w = n_rows // nw
    n_ir = rows_per_w // 128
    ch = 256
    n_ch = rows_per_w // ch
    nbuf = 3
    lag = 1
    n_grp = (n_ch + nbuf - 1) // nbuf + 1
    mesh = plsc.VectorSubcoreMesh(core_axis_name="c", subcore_axis_name="s")

    @functools.partial(
        pl.kernel, mesh=mesh,
        out_type=jax.ShapeDtypeStruct((n_rows, DIM_MODEL), jnp.float32),
        scratch_types=[
            pltpu.VMEM((n_ir, 128), jnp.int32),
            pltpu.VMEM((nbuf, ch, DIM_MODEL), jnp.float32),
        ] + [pltpu.SemaphoreType.DMA] * (2 * nbuf),
    )
    def k(tbl_hbm, idx_hbm, out_hbm, idx_v, rows_v, *sems):
        sem_g = sems[:nbuf]
        sem_s = sems[nbuf:]
        wid = lax.axis_index("s") * info.num_cores + lax.axis_index("c")
        out0 = wid * rows_per_w

        pltpu.sync_copy(idx_hbm.at[pl.ds(wid * n_ir, n_ir), :], idx_v)

        def gather_descs(c, b):
            return [
                pltpu.make_async_copy(
                    tbl_hbm.at[idx_v.at[2 * c + j]],
                    rows_v.at[b].at[pl.ds(j * 128, 128), :],
                    sem_g[b])
                for j in range(2)
            ]

        def scatter_desc(c, b):
            return pltpu.make_async_copy(
                rows_v.at[b],
                out_hbm.at[pl.ds(out0 + c * ch, ch), :],
                sem_s[b])

        def group(i, carry):
            for b in range(nbuf):
                c = i * nbuf + b
                bs = (b + nbuf - lag) % nbuf

                @pl.when(jnp.logical_and(c >= nbuf, c - nbuf < n_ch))
                def _():
                    scatter_desc(c - nbuf, b).wait()

                @pl.when(c < n_ch)
                def _():
                    for d in gather_descs(c, b):
                        d.start()

                @pl.when(jnp.logical_and(c >= lag, c - lag < n_ch))
                def _():
                    for d in gather_descs(c - lag, bs):
                        d.wait()
                    scatter_desc(c - lag, bs).start()
            return carry

        lax.fori_loop(0, n_grp, group, 0)

    return k(tbl, idx2d)


def kernel(x, table, W2, b2, gamma, beta):
    B, L = x.shape
    t = _transform_table(table, W2, b2, gamma, beta)
    idx2d = x.reshape(-1, 128).astype(jnp.int32)
    out = _sc_gather(t, idx2d, B * L)
    return out.reshape(B, L, DIM_MODEL)
